# baseline (speedup 1.0000x reference)
import functools

import jax
import jax.numpy as jnp
from jax import lax
from jax.experimental import pallas as pl
from jax.experimental.pallas import tpu as pltpu
from jax.experimental.pallas import tpu_sc as plsc

NC = 2
NS = 16
NW = NC * NS
CH = 128


def _mm(x, w):
    return lax.dot_general(x, w, (((1,), (1,)), ((), ())),
                           preferred_element_type=jnp.float32)


def _proj4_body(x_ref, w1, w2, wb, wc, o1, o2, ob, oc):
    x = x_ref[...]
    o1[...] = _mm(x, w1[...])
    o2[...] = _mm(x, w2[...])
    ob[...] = _mm(x, wb[...])
    oc[...] = _mm(x, wc[...])


def _proj4(x, w1, w2, wb, wc, bm):
    m, k = x.shape
    f = w1.shape[0]
    wspec = pl.BlockSpec((f, k), lambda i: (0, 0))
    return pl.pallas_call(
        _proj4_body,
        grid=(m // bm,),
        in_specs=[pl.BlockSpec((bm, k), lambda i: (i, 0))] + [wspec] * 4,
        out_specs=[pl.BlockSpec((bm, f), lambda i: (i, 0))] * 4,
        out_shape=[jax.ShapeDtypeStruct((m, f), jnp.float32)] * 4,
    )(x, w1, w2, wb, wc)


def _premix_body(xs_ref, wv_ref, wm1, wmixsum, wself, wmsg, wm1n, bias,
                 q1, q2):
    xs = xs_ref[...]
    wv = wv_ref[...]
    x = wv[0, 0] * xs[0]
    for tt in range(1, xs.shape[0]):
        x = x + wv[0, tt] * xs[tt]
    b_mix = bias[0:1, :]
    b_self = bias[1:2, :]
    b_msg = bias[2:3, :]
    q1[...] = _mm(x, wm1[...])
    mix = _mm(x, wmixsum[...]) + b_mix
    x_up = _mm(x, wself[...]) + b_self + _mm(mix, wmsg[...]) + b_msg
    q2[...] = _mm(x_up, wm1n[...])


def _premix(xs, wv, wm1, wmixsum, wself, wmsg, wm1n, bias, bm):
    t, m, din = xs.shape
    f = wm1.shape[0]

    def full(shape):
        return pl.BlockSpec(shape, lambda i: tuple(0 for _ in shape))

    return pl.pallas_call(
        _premix_body,
        grid=(m // bm,),
        in_specs=[pl.BlockSpec((t, bm, din), lambda i: (0, i, 0)),
                  full(wv.shape), full(wm1.shape), full(wmixsum.shape),
                  full(wself.shape), full(wmsg.shape), full(wm1n.shape),
                  full(bias.shape)],
        out_specs=[pl.BlockSpec((bm, f), lambda i: (i, 0))] * 2,
        out_shape=[jax.ShapeDtypeStruct((m, f), jnp.float32)] * 2,
    )(xs, wv, wm1, wmixsum, wself, wmsg, wm1n, bias)


def _agg_update_body(x_ref, p0, p1, deg, p2, pb2, pc2,
                     wsi, wmi, bias, *rest):
    d_i = deg[0, :, 0:1] + deg[1, :, 0:1]
    d_b = deg[0, :, 1:2] + deg[1, :, 1:2]
    d_c = deg[0, :, 2:3] + deg[1, :, 2:3]
    b_ii = bias[0:1, :]
    b_bi = bias[1:2, :]
    b_ci = bias[2:3, :]
    b_si = bias[3:4, :]
    b_mi = bias[4:5, :]
    sums = (p0[...] + p1[...]
            + d_i * (p2[...] + b_ii)
            + d_b * (pb2[...] + b_bi)
            + d_c * (pc2[...] + b_ci))
    agg = sums / jnp.maximum(d_i + d_b + d_c, 1.0)
    dx = _mm(x_ref[...], wsi[...]) + b_si + _mm(agg, wmi[...]) + b_mi
    if len(rest) == 1:
        rest[0][...] = dx
    else:
        w1, w2, wb, wc, dx_o, o1, o2, ob, oc = rest
        dx_o[...] = dx
        o1[...] = _mm(dx, w1[...])
        o2[...] = _mm(dx, w2[...])
        ob[...] = _mm(dx, wb[...])
        oc[...] = _mm(dx, wc[...])


def _agg_update(x, p0, p1, deg, p2, pb2, pc2, wsi, wmi, bias,
                next_ws, bm):
    m, d = x.shape
    f = wmi.shape[0]

    def full(shape):
        return pl.BlockSpec(shape, lambda i: tuple(0 for _ in shape))

    row = pl.BlockSpec((bm, d), lambda i: (i, 0))
    degspec = pl.BlockSpec((2, bm, deg.shape[2]), lambda i: (0, i, 0))
    n_out = 5 if next_ws else 1
    extra = [full(w.shape) for w in (next_ws or [])]
    outs = pl.pallas_call(
        _agg_update_body,
        grid=(m // bm,),
        in_specs=[row, row, row, degspec, row, row, row,
                  full(wsi.shape), full(wmi.shape), full(bias.shape)] + extra,
        out_specs=[pl.BlockSpec((bm, f), lambda i: (i, 0))] * n_out,
        out_shape=[jax.ShapeDtypeStruct((m, f), jnp.float32)] * n_out,
    )(x, p0, p1, deg, p2, pb2, pc2, wsi, wmi, bias, *(next_ws or []))
    return outs


def _sc_segsum(table, src3, tgt3, zeros, acc_r, g):
    dm = table.shape[1]
    rpt = acc_r // NS
    mesh = plsc.VectorSubcoreMesh(core_axis_name="c", subcore_axis_name="s")

    @functools.partial(
        pl.kernel,
        out_type=jax.ShapeDtypeStruct((NC * acc_r, dm), jnp.float32),
        mesh=mesh,
        scratch_types=[
            pltpu.VMEM((g * CH,), jnp.int32),
            pltpu.VMEM((g, CH), jnp.int32),
            pltpu.VMEM((CH, dm), jnp.float32),
            pltpu.VMEM_SHARED((acc_r, dm), jnp.float32),
            pltpu.SemaphoreType.DMA,
        ])
    def k(table_h, src_h, tgt_h, zeros_h, out_h,
          idxs_v, idxt_v, rows_v, acc_sh, sem):
        c = lax.axis_index("c")
        s = lax.axis_index("s")
        w = s * NC + c
        pltpu.sync_copy(zeros_h.at[pl.ds(s * rpt, rpt)],
                        acc_sh.at[pl.ds(s * rpt, rpt)])
        pltpu.sync_copy(src_h.at[w], idxs_v)
        pltpu.sync_copy(tgt_h.at[w], idxt_v)
        plsc.subcore_barrier()

        def body(gi, carry):
            pltpu.async_copy(
                table_h.at[idxs_v.at[pl.ds(gi * CH, CH)]], rows_v, sem).wait()
            pltpu.sync_copy(rows_v, acc_sh.at[idxt_v.at[gi]], add=True)
            return carry

        lax.fori_loop(0, g, body, 0)
        plsc.subcore_barrier()
        pltpu.sync_copy(acc_sh.at[pl.ds(s * rpt, rpt)],
                        out_h.at[pl.ds(c * acc_r + s * rpt, rpt)])

    return k(table, src3, tgt3, zeros)


def kernel(t, x_int_t, X_bound, U, edge_index_int, edge_index_bound,
           edge_index_ctrl, timestamps, boundary_node_index,
           control_node_index, params):
    f32 = jnp.float32
    i32 = jnp.int32
    xi = x_int_t
    n, d = xi.shape
    nt = timestamps.shape[0]
    e_b = X_bound.shape[1]
    e_c = U.shape[1]
    dc = U.shape[2]
    e_int = edge_index_int.shape[1]
    mp1, mp2 = params["mp1"], params["mp2"]
    dm = mp1["message_int_int"]["W"].shape[0]
    dh = mp1["interior_self_W"]["W"].shape[0]

    tn = jnp.reshape(t, ())
    ts = timestamps
    mask = (ts[:-1] <= tn) & (tn <= ts[1:])
    k = jnp.max(jnp.where(mask, jnp.arange(nt - 1), -1))
    k = jnp.maximum(k, 0)
    s = (tn - ts[k]) / (ts[k + 1] - ts[k])
    over = tn > ts[-1]
    wv = jnp.zeros((nt,), f32).at[k].add(1.0 - s).at[k + 1].add(s)
    wv = jnp.where(over, jnp.zeros((nt,), f32).at[-1].set(1.0), wv)
    wv = wv.reshape(1, nt)

    acc_r = ((n // (NS * 8)) + 1) * NS * 8
    e_tot = e_int + e_b + e_c
    g = -(-e_tot // (NW * CH))
    g = g + (g % 2)
    pad = NW * g * CH - e_tot
    isrc, itgt = edge_index_int[0], edge_index_int[1]
    btgt = edge_index_bound[1]
    ctgt = edge_index_ctrl[1]
    src_all = jnp.concatenate([
        isrc, n + jnp.arange(e_b, dtype=i32), n + e_b + jnp.arange(e_c, dtype=i32),
        jnp.zeros((pad,), i32)]).reshape(NW, g * CH)
    tgt_all = jnp.concatenate([
        itgt, btgt, ctgt, jnp.full((pad,), n, i32)]).reshape(NW, g, CH)

    zeros_seg = jnp.zeros((acc_r, dm), f32)

    rep = 2048
    class_table = jnp.concatenate([
        jnp.repeat(jnp.zeros((3, dm), f32)
                   .at[0, 0].set(1.0).at[1, 1].set(1.0).at[2, 2].set(1.0),
                   rep, axis=0),
        jnp.zeros((8, dm), f32)])
    src_deg = jnp.concatenate([
        jnp.arange(e_int, dtype=i32) % rep,
        rep + jnp.arange(e_b, dtype=i32) % rep,
        2 * rep + jnp.arange(e_c, dtype=i32) % rep,
        jnp.full((pad,), 3 * rep, i32)]).reshape(NW, g * CH)
    deg = _sc_segsum(class_table, src_deg, tgt_all, zeros_seg, acc_r, g)
    deg = deg.reshape(NC, acc_r, dm)[:, 0:n, :]

    bm = 1000

    def pack_bias(*bs):
        rows = jnp.stack([b.astype(f32) for b in bs])
        return jnp.concatenate(
            [rows, jnp.zeros((8 - rows.shape[0], rows.shape[1]), f32)])

    wii1 = mp1["message_int_int"]["W"]
    wbi1 = mp1["message_bound_int"]["W"]
    wci1 = mp1["message_ctrl_int"]["W"]
    p1_1, p2_1, pb2_1, pc2_1 = _proj4(
        xi, wii1[:, :d], wii1[:, d:], wbi1[:, d:], wci1[:, dc:], bm)

    wbb = mp1["message_bound_bound"]["W"]
    qb1, qb2 = _premix(
        X_bound, wv, wbi1[:, :d], wbb[:, :d] + wbb[:, d:],
        mp1["boundary_self_W"]["W"], mp1["boundary_msg_W"]["W"],
        mp2["message_bound_int"]["W"][:, :dh],
        pack_bias(mp1["message_bound_bound"]["b"],
                  mp1["boundary_self_W"]["b"], mp1["boundary_msg_W"]["b"]),
        bm)
    wcc = mp1["message_ctrl_ctrl"]["W"]
    qc1, qc2 = _premix(
        U, wv, wci1[:, :dc], wcc[:, :dc] + wcc[:, dc:],
        mp1["control_self_W"]["W"], mp1["control_msg_W"]["W"],
        mp2["message_ctrl_int"]["W"][:, :dh],
        pack_bias(mp1["message_ctrl_ctrl"]["b"],
                  mp1["control_self_W"]["b"], mp1["control_msg_W"]["b"]),
        bm)

    t1 = jnp.concatenate([p1_1, qb1, qc1], axis=0)
    seg1 = _sc_segsum(t1, src_all, tgt_all, zeros_seg, acc_r, g)
    seg1 = seg1.reshape(NC, acc_r, dm)

    wii2 = mp2["message_int_int"]["W"]
    wbi2 = mp2["message_bound_int"]["W"]
    wci2 = mp2["message_ctrl_int"]["W"]
    dx, p1_2, p2_2, pb2_2, pc2_2 = _agg_update(
        xi, seg1[0, :n], seg1[1, :n], deg, p2_1, pb2_1, pc2_1,
        mp1["interior_self_W"]["W"], mp1["interior_msg_W"]["W"],
        pack_bias(mp1["message_int_int"]["b"], mp1["message_bound_int"]["b"],
                  mp1["message_ctrl_int"]["b"], mp1["interior_self_W"]["b"],
                  mp1["interior_msg_W"]["b"]),
        [wii2[:, :dh], wii2[:, dh:], wbi2[:, dh:], wci2[:, dh:]], bm)

    t2 = jnp.concatenate([p1_2, qb2, qc2], axis=0)
    seg2 = _sc_segsum(t2, src_all, tgt_all, zeros_seg, acc_r, g)
    seg2 = seg2.reshape(NC, acc_r, dm)

    (out,) = _agg_update(
        dx, seg2[0, :n], seg2[1, :n], deg, p2_2, pb2_2, pc2_2,
        mp2["interior_self_W"]["W"], mp2["interior_msg_W"]["W"],
        pack_bias(mp2["message_int_int"]["b"], mp2["message_bound_int"]["b"],
                  mp2["message_ctrl_int"]["b"], mp2["interior_self_W"]["b"],
                  mp2["interior_msg_W"]["b"]),
        None, bm)

    return (out,
            jnp.zeros_like(X_bound),
            jnp.zeros_like(U),
            jnp.zeros_like(edge_index_int),
            jnp.zeros_like(edge_index_bound),
            jnp.zeros_like(edge_index_ctrl),
            jnp.zeros_like(timestamps))


# speedup vs baseline: 1.4280x; 1.4280x over previous
import functools

import jax
import jax.numpy as jnp
from jax import lax
from jax.experimental import pallas as pl
from jax.experimental.pallas import tpu as pltpu
from jax.experimental.pallas import tpu_sc as plsc

NC = 2
NS = 16
NW = NC * NS
CH = 128


def _mm(x, w):
    return lax.dot_general(x, w, (((1,), (1,)), ((), ())),
                           preferred_element_type=jnp.float32)


def _proj4_body(x_ref, w1, w2, wb, wc, o1, o2, ob, oc):
    x = x_ref[...]
    o1[...] = _mm(x, w1[...])
    o2[...] = _mm(x, w2[...])
    ob[...] = _mm(x, wb[...])
    oc[...] = _mm(x, wc[...])


def _proj4(x, w1, w2, wb, wc, bm):
    m, k = x.shape
    f = w1.shape[0]
    wspec = pl.BlockSpec((f, k), lambda i: (0, 0))
    return pl.pallas_call(
        _proj4_body,
        grid=(m // bm,),
        in_specs=[pl.BlockSpec((bm, k), lambda i: (i, 0))] + [wspec] * 4,
        out_specs=[pl.BlockSpec((bm, f), lambda i: (i, 0))] * 4,
        out_shape=[jax.ShapeDtypeStruct((m, f), jnp.float32)] * 4,
    )(x, w1, w2, wb, wc)


def _premix_body(xs_ref, wv_ref, wm1, wmixsum, wself, wmsg, wm1n, bias,
                 q1, q2):
    xs = xs_ref[...]
    wv = wv_ref[...]
    x = wv[0, 0] * xs[0]
    for tt in range(1, xs.shape[0]):
        x = x + wv[0, tt] * xs[tt]
    b_mix = bias[0:1, :]
    b_self = bias[1:2, :]
    b_msg = bias[2:3, :]
    q1[...] = _mm(x, wm1[...])
    mix = _mm(x, wmixsum[...]) + b_mix
    x_up = _mm(x, wself[...]) + b_self + _mm(mix, wmsg[...]) + b_msg
    q2[...] = _mm(x_up, wm1n[...])


def _premix(xs, wv, wm1, wmixsum, wself, wmsg, wm1n, bias, bm):
    t, m, din = xs.shape
    f = wm1.shape[0]

    def full(shape):
        return pl.BlockSpec(shape, lambda i: tuple(0 for _ in shape))

    return pl.pallas_call(
        _premix_body,
        grid=(m // bm,),
        in_specs=[pl.BlockSpec((t, bm, din), lambda i: (0, i, 0)),
                  full(wv.shape), full(wm1.shape), full(wmixsum.shape),
                  full(wself.shape), full(wmsg.shape), full(wm1n.shape),
                  full(bias.shape)],
        out_specs=[pl.BlockSpec((bm, f), lambda i: (i, 0))] * 2,
        out_shape=[jax.ShapeDtypeStruct((m, f), jnp.float32)] * 2,
    )(xs, wv, wm1, wmixsum, wself, wmsg, wm1n, bias)


def _agg_update_body(x_ref, p0, p1, deg, p2, pb2, pc2,
                     wsi, wmi, bias, *rest):
    d_i = deg[0, :, 0:1] + deg[1, :, 0:1]
    d_b = deg[0, :, 1:2] + deg[1, :, 1:2]
    d_c = deg[0, :, 2:3] + deg[1, :, 2:3]
    b_ii = bias[0:1, :]
    b_bi = bias[1:2, :]
    b_ci = bias[2:3, :]
    b_si = bias[3:4, :]
    b_mi = bias[4:5, :]
    sums = (p0[...] + p1[...]
            + d_i * (p2[...] + b_ii)
            + d_b * (pb2[...] + b_bi)
            + d_c * (pc2[...] + b_ci))
    agg = sums / jnp.maximum(d_i + d_b + d_c, 1.0)
    dx = _mm(x_ref[...], wsi[...]) + b_si + _mm(agg, wmi[...]) + b_mi
    if len(rest) == 1:
        rest[0][...] = dx
    else:
        w1, w2, wb, wc, dx_o, o1, o2, ob, oc = rest
        dx_o[...] = dx
        o1[...] = _mm(dx, w1[...])
        o2[...] = _mm(dx, w2[...])
        ob[...] = _mm(dx, wb[...])
        oc[...] = _mm(dx, wc[...])


def _agg_update(x, p0, p1, deg, p2, pb2, pc2, wsi, wmi, bias,
                next_ws, bm):
    m, d = x.shape
    f = wmi.shape[0]

    def full(shape):
        return pl.BlockSpec(shape, lambda i: tuple(0 for _ in shape))

    row = pl.BlockSpec((bm, d), lambda i: (i, 0))
    degspec = pl.BlockSpec((2, bm, deg.shape[2]), lambda i: (0, i, 0))
    n_out = 5 if next_ws else 1
    extra = [full(w.shape) for w in (next_ws or [])]
    outs = pl.pallas_call(
        _agg_update_body,
        grid=(m // bm,),
        in_specs=[row, row, row, degspec, row, row, row,
                  full(wsi.shape), full(wmi.shape), full(bias.shape)] + extra,
        out_specs=[pl.BlockSpec((bm, f), lambda i: (i, 0))] * n_out,
        out_shape=[jax.ShapeDtypeStruct((m, f), jnp.float32)] * n_out,
    )(x, p0, p1, deg, p2, pb2, pc2, wsi, wmi, bias, *(next_ws or []))
    return outs


def _sc_segsum(table, src_f, tgt_f, zeros, acc_r, g0, g1):
    dm = table.shape[1]
    rpt = acc_r // NS
    mesh = plsc.VectorSubcoreMesh(core_axis_name="c", subcore_axis_name="s")

    @functools.partial(
        pl.kernel,
        out_type=jax.ShapeDtypeStruct((NC * acc_r, dm), jnp.float32),
        mesh=mesh,
        scratch_types=[
            pltpu.VMEM((CH,), jnp.int32),
            pltpu.VMEM((CH,), jnp.int32),
            pltpu.VMEM((CH, dm), jnp.float32),
            pltpu.VMEM_SHARED((acc_r, dm), jnp.float32),
            pltpu.SemaphoreType.DMA,
        ])
    def k(table_h, src_h, tgt_h, zeros_h, out_h,
          idxs_v, idxt_v, rows_v, acc_sh, sem):
        c = lax.axis_index("c")
        s = lax.axis_index("s")
        my_g = jnp.where(c == 0, g0, g1)
        base = jnp.where(c == 0, s * g0, 16 * g0 + s * g1)
        pltpu.sync_copy(zeros_h.at[pl.ds(s * rpt, rpt)],
                        acc_sh.at[pl.ds(s * rpt, rpt)])
        plsc.subcore_barrier()

        def body(gi, carry):
            off = (base + gi) * CH
            pltpu.sync_copy(src_h.at[pl.ds(off, CH)], idxs_v)
            pltpu.sync_copy(tgt_h.at[pl.ds(off, CH)], idxt_v)
            pltpu.async_copy(table_h.at[idxs_v], rows_v, sem).wait()
            pltpu.sync_copy(rows_v, acc_sh.at[idxt_v], add=True)
            return carry

        lax.fori_loop(0, my_g, body, 0)
        plsc.subcore_barrier()
        pltpu.sync_copy(acc_sh.at[pl.ds(s * rpt, rpt)],
                        out_h.at[pl.ds(c * acc_r + s * rpt, rpt)])

    return k(table, src_f, tgt_f, zeros)


def kernel(t, x_int_t, X_bound, U, edge_index_int, edge_index_bound,
           edge_index_ctrl, timestamps, boundary_node_index,
           control_node_index, params):
    f32 = jnp.float32
    i32 = jnp.int32
    xi = x_int_t
    n, d = xi.shape
    nt = timestamps.shape[0]
    e_b = X_bound.shape[1]
    e_c = U.shape[1]
    dc = U.shape[2]
    e_int = edge_index_int.shape[1]
    mp1, mp2 = params["mp1"], params["mp2"]
    dm = mp1["message_int_int"]["W"].shape[0]
    dh = mp1["interior_self_W"]["W"].shape[0]

    tn = jnp.reshape(t, ())
    ts = timestamps
    mask = (ts[:-1] <= tn) & (tn <= ts[1:])
    k = jnp.max(jnp.where(mask, jnp.arange(nt - 1), -1))
    k = jnp.maximum(k, 0)
    s = (tn - ts[k]) / (ts[k + 1] - ts[k])
    over = tn > ts[-1]
    wv = jnp.zeros((nt,), f32).at[k].add(1.0 - s).at[k + 1].add(s)
    wv = jnp.where(over, jnp.zeros((nt,), f32).at[-1].set(1.0), wv)
    wv = wv.reshape(1, nt)

    acc_r = ((n // (NS * 8)) + 1) * NS * 8
    e_tot = e_int + e_b + e_c
    ct = -(-e_tot // CH)
    ct = -(-ct // 16) * 16
    pad = ct * CH - e_tot
    g0 = max(1, int(ct * 0.41 / 16 + 0.5))
    g1 = ct // 16 - g0
    isrc, itgt = edge_index_int[0], edge_index_int[1]
    btgt = edge_index_bound[1]
    ctgt = edge_index_ctrl[1]
    src_all = jnp.concatenate([
        isrc, n + jnp.arange(e_b, dtype=i32), n + e_b + jnp.arange(e_c, dtype=i32),
        jnp.zeros((pad,), i32)])
    tgt_all = jnp.concatenate([
        itgt, btgt, ctgt, jnp.full((pad,), n, i32)])

    zeros_seg = jnp.zeros((acc_r, dm), f32)

    rep = 2048
    class_table = jnp.concatenate([
        jnp.repeat(jnp.zeros((3, dm), f32)
                   .at[0, 0].set(1.0).at[1, 1].set(1.0).at[2, 2].set(1.0),
                   rep, axis=0),
        jnp.zeros((8, dm), f32)])
    src_deg = jnp.concatenate([
        jnp.arange(e_int, dtype=i32) % rep,
        rep + jnp.arange(e_b, dtype=i32) % rep,
        2 * rep + jnp.arange(e_c, dtype=i32) % rep,
        jnp.full((pad,), 3 * rep, i32)])
    deg = _sc_segsum(class_table, src_deg, tgt_all, zeros_seg, acc_r, g0, g1)
    deg = deg.reshape(NC, acc_r, dm)[:, 0:n, :]

    bm = 1000

    def pack_bias(*bs):
        rows = jnp.stack([b.astype(f32) for b in bs])
        return jnp.concatenate(
            [rows, jnp.zeros((8 - rows.shape[0], rows.shape[1]), f32)])

    wii1 = mp1["message_int_int"]["W"]
    wbi1 = mp1["message_bound_int"]["W"]
    wci1 = mp1["message_ctrl_int"]["W"]
    p1_1, p2_1, pb2_1, pc2_1 = _proj4(
        xi, wii1[:, :d], wii1[:, d:], wbi1[:, d:], wci1[:, dc:], bm)

    wbb = mp1["message_bound_bound"]["W"]
    qb1, qb2 = _premix(
        X_bound, wv, wbi1[:, :d], wbb[:, :d] + wbb[:, d:],
        mp1["boundary_self_W"]["W"], mp1["boundary_msg_W"]["W"],
        mp2["message_bound_int"]["W"][:, :dh],
        pack_bias(mp1["message_bound_bound"]["b"],
                  mp1["boundary_self_W"]["b"], mp1["boundary_msg_W"]["b"]),
        bm)
    wcc = mp1["message_ctrl_ctrl"]["W"]
    qc1, qc2 = _premix(
        U, wv, wci1[:, :dc], wcc[:, :dc] + wcc[:, dc:],
        mp1["control_self_W"]["W"], mp1["control_msg_W"]["W"],
        mp2["message_ctrl_int"]["W"][:, :dh],
        pack_bias(mp1["message_ctrl_ctrl"]["b"],
                  mp1["control_self_W"]["b"], mp1["control_msg_W"]["b"]),
        bm)

    t1 = jnp.concatenate([p1_1, qb1, qc1], axis=0)
    seg1 = _sc_segsum(t1, src_all, tgt_all, zeros_seg, acc_r, g0, g1)
    seg1 = seg1.reshape(NC, acc_r, dm)

    wii2 = mp2["message_int_int"]["W"]
    wbi2 = mp2["message_bound_int"]["W"]
    wci2 = mp2["message_ctrl_int"]["W"]
    dx, p1_2, p2_2, pb2_2, pc2_2 = _agg_update(
        xi, seg1[0, :n], seg1[1, :n], deg, p2_1, pb2_1, pc2_1,
        mp1["interior_self_W"]["W"], mp1["interior_msg_W"]["W"],
        pack_bias(mp1["message_int_int"]["b"], mp1["message_bound_int"]["b"],
                  mp1["message_ctrl_int"]["b"], mp1["interior_self_W"]["b"],
                  mp1["interior_msg_W"]["b"]),
        [wii2[:, :dh], wii2[:, dh:], wbi2[:, dh:], wci2[:, dh:]], bm)

    t2 = jnp.concatenate([p1_2, qb2, qc2], axis=0)
    seg2 = _sc_segsum(t2, src_all, tgt_all, zeros_seg, acc_r, g0, g1)
    seg2 = seg2.reshape(NC, acc_r, dm)

    (out,) = _agg_update(
        dx, seg2[0, :n], seg2[1, :n], deg, p2_2, pb2_2, pc2_2,
        mp2["interior_self_W"]["W"], mp2["interior_msg_W"]["W"],
        pack_bias(mp2["message_int_int"]["b"], mp2["message_bound_int"]["b"],
                  mp2["message_ctrl_int"]["b"], mp2["interior_self_W"]["b"],
                  mp2["interior_msg_W"]["b"]),
        None, bm)

    return (out,
            jnp.zeros_like(X_bound),
            jnp.zeros_like(U),
            jnp.zeros_like(edge_index_int),
            jnp.zeros_like(edge_index_bound),
            jnp.zeros_like(edge_index_ctrl),
            jnp.zeros_like(timestamps))


# speedup vs baseline: 1.6142x; 1.1304x over previous
import functools

import jax
import jax.numpy as jnp
from jax import lax
from jax.experimental import pallas as pl
from jax.experimental.pallas import tpu as pltpu
from jax.experimental.pallas import tpu_sc as plsc

NC = 2
NS = 16
NW = NC * NS
CH = 128


def _mm(x, w):
    return lax.dot_general(x, w, (((1,), (1,)), ((), ())),
                           preferred_element_type=jnp.float32)


def _proj4_body(x_ref, w1, w2, wb, wc, o1, o2, ob, oc):
    x = x_ref[...]
    o1[...] = _mm(x, w1[...])
    o2[...] = _mm(x, w2[...])
    ob[...] = _mm(x, wb[...])
    oc[...] = _mm(x, wc[...])


def _proj4(x, w1, w2, wb, wc, bm):
    m, k = x.shape
    f = w1.shape[0]
    wspec = pl.BlockSpec((f, k), lambda i: (0, 0))
    return pl.pallas_call(
        _proj4_body,
        grid=(m // bm,),
        in_specs=[pl.BlockSpec((bm, k), lambda i: (i, 0))] + [wspec] * 4,
        out_specs=[pl.BlockSpec((bm, f), lambda i: (i, 0))] * 4,
        out_shape=[jax.ShapeDtypeStruct((m, f), jnp.float32)] * 4,
    )(x, w1, w2, wb, wc)


def _premix_body(xs_ref, wv_ref, wm1, wmixsum, wself, wmsg, wm1n, bias,
                 q1, q2):
    xs = xs_ref[...]
    wv = wv_ref[...]
    x = wv[0, 0] * xs[0]
    for tt in range(1, xs.shape[0]):
        x = x + wv[0, tt] * xs[tt]
    b_mix = bias[0:1, :]
    b_self = bias[1:2, :]
    b_msg = bias[2:3, :]
    q1[...] = _mm(x, wm1[...])
    mix = _mm(x, wmixsum[...]) + b_mix
    x_up = _mm(x, wself[...]) + b_self + _mm(mix, wmsg[...]) + b_msg
    q2[...] = _mm(x_up, wm1n[...])


def _premix(xs, wv, wm1, wmixsum, wself, wmsg, wm1n, bias, bm):
    t, m, din = xs.shape
    f = wm1.shape[0]

    def full(shape):
        return pl.BlockSpec(shape, lambda i: tuple(0 for _ in shape))

    return pl.pallas_call(
        _premix_body,
        grid=(m // bm,),
        in_specs=[pl.BlockSpec((t, bm, din), lambda i: (0, i, 0)),
                  full(wv.shape), full(wm1.shape), full(wmixsum.shape),
                  full(wself.shape), full(wmsg.shape), full(wm1n.shape),
                  full(bias.shape)],
        out_specs=[pl.BlockSpec((bm, f), lambda i: (i, 0))] * 2,
        out_shape=[jax.ShapeDtypeStruct((m, f), jnp.float32)] * 2,
    )(xs, wv, wm1, wmixsum, wself, wmsg, wm1n, bias)


def _agg_update_body(x_ref, p0, p1, deg, p2, pb2, pc2,
                     wsi, wmi, bias, *rest):
    d_i = deg[0, :, 0:1] + deg[1, :, 0:1]
    d_b = deg[0, :, 1:2] + deg[1, :, 1:2]
    d_c = deg[0, :, 2:3] + deg[1, :, 2:3]
    b_ii = bias[0:1, :]
    b_bi = bias[1:2, :]
    b_ci = bias[2:3, :]
    b_si = bias[3:4, :]
    b_mi = bias[4:5, :]
    sums = (p0[...] + p1[...]
            + d_i * (p2[...] + b_ii)
            + d_b * (pb2[...] + b_bi)
            + d_c * (pc2[...] + b_ci))
    agg = sums / jnp.maximum(d_i + d_b + d_c, 1.0)
    dx = _mm(x_ref[...], wsi[...]) + b_si + _mm(agg, wmi[...]) + b_mi
    if len(rest) == 1:
        rest[0][...] = dx
    else:
        w1, w2, wb, wc, dx_o, o1, o2, ob, oc = rest
        dx_o[...] = dx
        o1[...] = _mm(dx, w1[...])
        o2[...] = _mm(dx, w2[...])
        ob[...] = _mm(dx, wb[...])
        oc[...] = _mm(dx, wc[...])


def _agg_update(x, p0, p1, deg, p2, pb2, pc2, wsi, wmi, bias,
                next_ws, bm):
    m, d = x.shape
    f = wmi.shape[0]

    def full(shape):
        return pl.BlockSpec(shape, lambda i: tuple(0 for _ in shape))

    row = pl.BlockSpec((bm, d), lambda i: (i, 0))
    degspec = pl.BlockSpec((2, bm, deg.shape[2]), lambda i: (0, i, 0))
    n_out = 5 if next_ws else 1
    extra = [full(w.shape) for w in (next_ws or [])]
    outs = pl.pallas_call(
        _agg_update_body,
        grid=(m // bm,),
        in_specs=[row, row, row, degspec, row, row, row,
                  full(wsi.shape), full(wmi.shape), full(bias.shape)] + extra,
        out_specs=[pl.BlockSpec((bm, f), lambda i: (i, 0))] * n_out,
        out_shape=[jax.ShapeDtypeStruct((m, f), jnp.float32)] * n_out,
    )(x, p0, p1, deg, p2, pb2, pc2, wsi, wmi, bias, *(next_ws or []))
    return outs


def _sc_segsum(table, src_f, tgt_f, zeros, acc_r, g0, g1):
    dm = table.shape[1]
    rpt = acc_r // NS
    mesh = plsc.VectorSubcoreMesh(core_axis_name="c", subcore_axis_name="s")

    @functools.partial(
        pl.kernel,
        out_type=jax.ShapeDtypeStruct((NC * acc_r, dm), jnp.float32),
        mesh=mesh,
        scratch_types=[
            pltpu.VMEM((CH,), jnp.int32),
            pltpu.VMEM((CH,), jnp.int32),
            pltpu.VMEM((CH, dm), jnp.float32),
            pltpu.VMEM_SHARED((acc_r, dm), jnp.float32),
            pltpu.SemaphoreType.DMA,
        ])
    def k(table_h, src_h, tgt_h, zeros_h, out_h,
          idxs_v, idxt_v, rows_v, acc_sh, sem):
        c = lax.axis_index("c")
        s = lax.axis_index("s")
        my_g = jnp.where(c == 0, g0, g1)
        base = jnp.where(c == 0, s * g0, 16 * g0 + s * g1)
        pltpu.sync_copy(zeros_h.at[pl.ds(s * rpt, rpt)],
                        acc_sh.at[pl.ds(s * rpt, rpt)])
        plsc.subcore_barrier()

        def body(gi, carry):
            off = (base + gi) * CH
            pltpu.sync_copy(src_h.at[pl.ds(off, CH)], idxs_v)
            pltpu.sync_copy(tgt_h.at[pl.ds(off, CH)], idxt_v)
            pltpu.async_copy(table_h.at[idxs_v], rows_v, sem).wait()
            pltpu.sync_copy(rows_v, acc_sh.at[idxt_v], add=True)
            return carry

        lax.fori_loop(0, my_g, body, 0)
        plsc.subcore_barrier()
        pltpu.sync_copy(acc_sh.at[pl.ds(s * rpt, rpt)],
                        out_h.at[pl.ds(c * acc_r + s * rpt, rpt)])

    return k(table, src_f, tgt_f, zeros)


def kernel(t, x_int_t, X_bound, U, edge_index_int, edge_index_bound,
           edge_index_ctrl, timestamps, boundary_node_index,
           control_node_index, params):
    f32 = jnp.float32
    i32 = jnp.int32
    xi = x_int_t
    n, d = xi.shape
    nt = timestamps.shape[0]
    e_b = X_bound.shape[1]
    e_c = U.shape[1]
    dc = U.shape[2]
    e_int = edge_index_int.shape[1]
    mp1, mp2 = params["mp1"], params["mp2"]
    dm = mp1["message_int_int"]["W"].shape[0]
    dh = mp1["interior_self_W"]["W"].shape[0]

    tn = jnp.reshape(t, ())
    ts = timestamps
    mask = (ts[:-1] <= tn) & (tn <= ts[1:])
    k = jnp.max(jnp.where(mask, jnp.arange(nt - 1), -1))
    k = jnp.maximum(k, 0)
    s = (tn - ts[k]) / (ts[k + 1] - ts[k])
    over = tn > ts[-1]
    wv = jnp.zeros((nt,), f32).at[k].add(1.0 - s).at[k + 1].add(s)
    wv = jnp.where(over, jnp.zeros((nt,), f32).at[-1].set(1.0), wv)
    wv = wv.reshape(1, nt)

    acc_r = ((n // (NS * 8)) + 1) * NS * 8
    e_tot = e_int + e_b + e_c
    ct = -(-e_tot // CH)
    ct = -(-ct // 16) * 16
    pad = ct * CH - e_tot
    g0 = max(1, int(ct * 0.52 / 16 + 0.5))
    g1 = ct // 16 - g0
    isrc, itgt = edge_index_int[0], edge_index_int[1]
    btgt = edge_index_bound[1]
    ctgt = edge_index_ctrl[1]
    src_all = jnp.concatenate([
        isrc, n + jnp.arange(e_b, dtype=i32), n + e_b + jnp.arange(e_c, dtype=i32),
        jnp.zeros((pad,), i32)])
    tgt_all = jnp.concatenate([
        itgt, btgt, ctgt, jnp.full((pad,), n, i32)])

    zeros_seg = jnp.zeros((acc_r, dm), f32)

    rep = 2048
    class_table = jnp.concatenate([
        jnp.repeat(jnp.zeros((3, dm), f32)
                   .at[0, 0].set(1.0).at[1, 1].set(1.0).at[2, 2].set(1.0),
                   rep, axis=0),
        jnp.zeros((8, dm), f32)])
    src_deg = jnp.concatenate([
        jnp.arange(e_int, dtype=i32) % rep,
        rep + jnp.arange(e_b, dtype=i32) % rep,
        2 * rep + jnp.arange(e_c, dtype=i32) % rep,
        jnp.full((pad,), 3 * rep, i32)])
    deg = _sc_segsum(class_table, src_deg, tgt_all, zeros_seg, acc_r, g0, g1)
    deg = deg.reshape(NC, acc_r, dm)[:, 0:n, :]

    bm = 1000

    def pack_bias(*bs):
        rows = jnp.stack([b.astype(f32) for b in bs])
        return jnp.concatenate(
            [rows, jnp.zeros((8 - rows.shape[0], rows.shape[1]), f32)])

    wii1 = mp1["message_int_int"]["W"]
    wbi1 = mp1["message_bound_int"]["W"]
    wci1 = mp1["message_ctrl_int"]["W"]
    p1_1, p2_1, pb2_1, pc2_1 = _proj4(
        xi, wii1[:, :d], wii1[:, d:], wbi1[:, d:], wci1[:, dc:], bm)

    wbb = mp1["message_bound_bound"]["W"]
    qb1, qb2 = _premix(
        X_bound, wv, wbi1[:, :d], wbb[:, :d] + wbb[:, d:],
        mp1["boundary_self_W"]["W"], mp1["boundary_msg_W"]["W"],
        mp2["message_bound_int"]["W"][:, :dh],
        pack_bias(mp1["message_bound_bound"]["b"],
                  mp1["boundary_self_W"]["b"], mp1["boundary_msg_W"]["b"]),
        bm)
    wcc = mp1["message_ctrl_ctrl"]["W"]
    qc1, qc2 = _premix(
        U, wv, wci1[:, :dc], wcc[:, :dc] + wcc[:, dc:],
        mp1["control_self_W"]["W"], mp1["control_msg_W"]["W"],
        mp2["message_ctrl_int"]["W"][:, :dh],
        pack_bias(mp1["message_ctrl_ctrl"]["b"],
                  mp1["control_self_W"]["b"], mp1["control_msg_W"]["b"]),
        bm)

    t1 = jnp.concatenate([p1_1, qb1, qc1], axis=0)
    seg1 = _sc_segsum(t1, src_all, tgt_all, zeros_seg, acc_r, g0, g1)
    seg1 = seg1.reshape(NC, acc_r, dm)

    wii2 = mp2["message_int_int"]["W"]
    wbi2 = mp2["message_bound_int"]["W"]
    wci2 = mp2["message_ctrl_int"]["W"]
    dx, p1_2, p2_2, pb2_2, pc2_2 = _agg_update(
        xi, seg1[0, :n], seg1[1, :n], deg, p2_1, pb2_1, pc2_1,
        mp1["interior_self_W"]["W"], mp1["interior_msg_W"]["W"],
        pack_bias(mp1["message_int_int"]["b"], mp1["message_bound_int"]["b"],
                  mp1["message_ctrl_int"]["b"], mp1["interior_self_W"]["b"],
                  mp1["interior_msg_W"]["b"]),
        [wii2[:, :dh], wii2[:, dh:], wbi2[:, dh:], wci2[:, dh:]], bm)

    t2 = jnp.concatenate([p1_2, qb2, qc2], axis=0)
    seg2 = _sc_segsum(t2, src_all, tgt_all, zeros_seg, acc_r, g0, g1)
    seg2 = seg2.reshape(NC, acc_r, dm)

    (out,) = _agg_update(
        dx, seg2[0, :n], seg2[1, :n], deg, p2_2, pb2_2, pc2_2,
        mp2["interior_self_W"]["W"], mp2["interior_msg_W"]["W"],
        pack_bias(mp2["message_int_int"]["b"], mp2["message_bound_int"]["b"],
                  mp2["message_ctrl_int"]["b"], mp2["interior_self_W"]["b"],
                  mp2["interior_msg_W"]["b"]),
        None, bm)

    return (out,
            jnp.zeros_like(X_bound),
            jnp.zeros_like(U),
            jnp.zeros_like(edge_index_int),
            jnp.zeros_like(edge_index_bound),
            jnp.zeros_like(edge_index_ctrl),
            jnp.zeros_like(timestamps))
